# Initial kernel scaffold; baseline (speedup 1.0000x reference)
import jax
import jax.numpy as jnp
from jax.experimental import pallas as pl


def kernel(x_user, x_item, edge_index_u2i, edge_index_i2u, pre_user_W, pre_user_b, pre_item_W, pre_item_b, post_user_W, post_user_b, post_item_W, post_item_b, l0_u2i_Wl, l0_u2i_Wr, l0_u2i_b, l0_i2u_Wl, l0_i2u_Wr, l0_i2u_b, l1_u2i_Wl, l1_u2i_Wr, l1_u2i_b, l1_i2u_Wl, l1_i2u_Wr, l1_i2u_b):
    raise NotImplementedError("write your pallas kernel here")


# speedup vs baseline: 3.2916x; 3.2916x over previous
import functools

import jax
import jax.numpy as jnp
from jax import lax
from jax.experimental import pallas as pl
from jax.experimental.pallas import tpu as pltpu
from jax.experimental.pallas import tpu_sc as plsc

N = 10000
D = 512
NCHUNK = 4
CW = 128
E = 78000
NC, NS = 2, 16
EB = 128
NBATCH = 39
E_PAD = NS * NBATCH * EB
TRASH = N
ACC_ROWS = 10240
ZROWS = ACC_ROWS // NS
OROWS = ACC_ROWS // NS

def _spmm_body(h_hbm, src_hbm, dst_hbm, zeros_hbm, out_hbm,
               acc, src_v, dst_v, gbuf, zbuf, sem):
  c = lax.axis_index("c")
  s = lax.axis_index("s")
  pltpu.sync_copy(src_hbm.at[s], src_v)
  pltpu.sync_copy(dst_hbm.at[s], dst_v)
  pltpu.sync_copy(zeros_hbm, zbuf)

  for it in range(NCHUNK // NC):
    chunk = c * (NCHUNK // NC) + it
    for k in range(ZROWS // 128):
      acc_rows = acc.at[pl.ds(s * ZROWS + k * 128, 128)]
      pltpu.sync_copy(zbuf, acc_rows)
    plsc.subcore_barrier()

    def body(j, carry):
      pltpu.async_copy(h_hbm.at[chunk].at[src_v.at[j]], gbuf, sem).wait()
      pltpu.sync_copy(gbuf, acc.at[dst_v.at[j]], add=True)
      return carry

    lax.fori_loop(0, NBATCH, body, 0)
    plsc.subcore_barrier()
    pltpu.sync_copy(acc.at[pl.ds(s * OROWS, OROWS)],
                    out_hbm.at[chunk].at[pl.ds(s * OROWS, OROWS)])
    plsc.subcore_barrier()


@functools.cache
def _make_spmm():
  return pl.kernel(
      _spmm_body,
      out_type=jax.ShapeDtypeStruct((NCHUNK, ACC_ROWS, CW), jnp.float32),
      mesh=plsc.VectorSubcoreMesh(core_axis_name="c", subcore_axis_name="s"),
      scratch_types=[
          pltpu.VMEM_SHARED((ACC_ROWS, CW), jnp.float32),
          pltpu.VMEM((NBATCH, EB), jnp.int32),
          pltpu.VMEM((NBATCH, EB), jnp.int32),
          pltpu.VMEM((EB, CW), jnp.float32),
          pltpu.VMEM((EB, CW), jnp.float32),
          pltpu.SemaphoreType.DMA,
      ],
  )


def _spmm(h, src_p, dst_p, zeros):
  return _make_spmm()(h, src_p, dst_p, zeros)


CNTW = 128


def _counts_body(dst2_hbm, zeros_hbm, ones_hbm, out_hbm, acc, dst_v, obuf, zbuf):
  c = lax.axis_index("c")
  s = lax.axis_index("s")
  pltpu.sync_copy(dst2_hbm.at[c].at[s], dst_v)
  pltpu.sync_copy(zeros_hbm, zbuf)
  pltpu.sync_copy(ones_hbm, obuf)
  for k in range(ZROWS // 128):
    pltpu.sync_copy(zbuf, acc.at[pl.ds(s * ZROWS + k * 128, 128)])
  plsc.subcore_barrier()

  def body(j, carry):
    pltpu.sync_copy(obuf, acc.at[dst_v.at[j]], add=True)
    return carry

  lax.fori_loop(0, NBATCH, body, 0)
  plsc.subcore_barrier()
  pltpu.sync_copy(acc.at[pl.ds(s * OROWS, OROWS)],
                  out_hbm.at[c].at[pl.ds(s * OROWS, OROWS)])


@functools.cache
def _make_counts():
  return pl.kernel(
      _counts_body,
      out_type=jax.ShapeDtypeStruct((NC, ACC_ROWS, CNTW), jnp.float32),
      mesh=plsc.VectorSubcoreMesh(core_axis_name="c", subcore_axis_name="s"),
      scratch_types=[
          pltpu.VMEM_SHARED((ACC_ROWS, CNTW), jnp.float32),
          pltpu.VMEM((NBATCH, EB), jnp.int32),
          pltpu.VMEM((EB, CNTW), jnp.float32),
          pltpu.VMEM((EB, CNTW), jnp.float32),
      ],
  )


def _counts(dst2, zeros, ones):
  return _make_counts()(dst2, zeros, ones)


BN = 2000
NBLK = N // BN
NEG_SLOPE = 0.01


def _leaky(y):
  return jnp.where(y >= 0, y, NEG_SLOPE * y)


def _mm(x, w):
  return lax.dot_general(x, w, (((1,), (1,)), ((), ())),
                         preferred_element_type=jnp.float32)


def _pre_body(x_ref, w_ref, b_ref, out_ref, acc):
  kc = pl.program_id(1)

  @pl.when(kc == 0)
  def _():
    acc[...] = jnp.zeros_like(acc)

  acc[...] += _mm(x_ref[...], w_ref[0])

  @pl.when(kc == NCHUNK - 1)
  def _():
    y = _leaky(acc[...] + b_ref[...])
    for cc in range(NCHUNK):
      out_ref[cc] = y[:, cc * CW:(cc + 1) * CW]


def _pre_mlp(x, w4, b):
  return pl.pallas_call(
      _pre_body,
      grid=(NBLK, NCHUNK),
      in_specs=[
          pl.BlockSpec((BN, CW), lambda nb, kc: (nb, kc)),
          pl.BlockSpec((1, D, CW), lambda nb, kc: (kc, 0, 0)),
          pl.BlockSpec((1, D), lambda nb, kc: (0, 0)),
      ],
      out_specs=pl.BlockSpec((NCHUNK, BN, CW), lambda nb, kc: (0, nb, 0)),
      out_shape=jax.ShapeDtypeStruct((NCHUNK, N, CW), jnp.float32),
      scratch_shapes=[pltpu.VMEM((BN, D), jnp.float32)],
  )(x, w4, b)


def _mid_body(agg_ref, cnt_ref, h_ref, wl_ref, wr_ref, b_ref, out_ref, acc):
  kc = pl.program_id(1)

  @pl.when(kc == 0)
  def _():
    acc[...] = jnp.zeros_like(acc)

  inv = 1.0 / jnp.maximum(cnt_ref[...], 1.0)
  acc[...] += _mm(agg_ref[0] * inv, wl_ref[0]) + _mm(h_ref[0], wr_ref[0])

  @pl.when(kc == NCHUNK - 1)
  def _():
    y = _leaky(acc[...] + b_ref[...])
    for cc in range(NCHUNK):
      out_ref[cc] = y[:, cc * CW:(cc + 1) * CW]


def _mid_mlp(agg, cnt, h, wl4, wr4, b):
  return pl.pallas_call(
      _mid_body,
      grid=(NBLK, NCHUNK),
      in_specs=[
          pl.BlockSpec((1, BN, CW), lambda nb, kc: (kc, nb, 0)),
          pl.BlockSpec((BN, 1), lambda nb, kc: (nb, 0)),
          pl.BlockSpec((1, BN, CW), lambda nb, kc: (kc, nb, 0)),
          pl.BlockSpec((1, D, CW), lambda nb, kc: (kc, 0, 0)),
          pl.BlockSpec((1, D, CW), lambda nb, kc: (kc, 0, 0)),
          pl.BlockSpec((1, D), lambda nb, kc: (0, 0)),
      ],
      out_specs=pl.BlockSpec((NCHUNK, BN, CW), lambda nb, kc: (0, nb, 0)),
      out_shape=jax.ShapeDtypeStruct((NCHUNK, N, CW), jnp.float32),
      scratch_shapes=[pltpu.VMEM((BN, D), jnp.float32)],
  )(agg, cnt, h, wl4, wr4, b)


def _post_body(h_ref, w_ref, b_ref, out_ref, acc):
  kc = pl.program_id(1)

  @pl.when(kc == 0)
  def _():
    acc[...] = jnp.zeros_like(acc)

  acc[...] += _mm(h_ref[0], w_ref[0])

  @pl.when(kc == NCHUNK - 1)
  def _():
    y = acc[...] + b_ref[...]
    nrm = jnp.maximum(jnp.sqrt(jnp.sum(y * y, axis=1, keepdims=True)), 1e-12)
    out_ref[...] = y / nrm


def _post_mlp(h, w4, b):
  return pl.pallas_call(
      _post_body,
      grid=(NBLK, NCHUNK),
      in_specs=[
          pl.BlockSpec((1, BN, CW), lambda nb, kc: (kc, nb, 0)),
          pl.BlockSpec((1, D, CW), lambda nb, kc: (kc, 0, 0)),
          pl.BlockSpec((1, D), lambda nb, kc: (0, 0)),
      ],
      out_specs=pl.BlockSpec((BN, D), lambda nb, kc: (nb, 0)),
      out_shape=jax.ShapeDtypeStruct((N, D), jnp.float32),
      scratch_shapes=[pltpu.VMEM((BN, D), jnp.float32)],
  )(h, w4, b)


def _chunk_w(w):
  return jnp.transpose(w.reshape(D, NCHUNK, CW), (1, 0, 2))


def _pad_edges(e):
  return jnp.concatenate(
      [e.astype(jnp.int32), jnp.full((E_PAD - E,), TRASH, jnp.int32)]
  ).reshape(NS, NBATCH, EB)


def kernel(x_user, x_item, edge_index_u2i, edge_index_i2u,
           pre_user_W, pre_user_b, pre_item_W, pre_item_b,
           post_user_W, post_user_b, post_item_W, post_item_b,
           l0_u2i_Wl, l0_u2i_Wr, l0_u2i_b,
           l0_i2u_Wl, l0_i2u_Wr, l0_i2u_b,
           l1_u2i_Wl, l1_u2i_Wr, l1_u2i_b,
           l1_i2u_Wl, l1_i2u_Wr, l1_i2u_b):
  zeros128 = jnp.zeros((EB, CW), jnp.float32)
  zeros16 = jnp.zeros((EB, CNTW), jnp.float32)
  ones16 = jnp.ones((EB, CNTW), jnp.float32)

  src_u2i = _pad_edges(edge_index_u2i[0])
  src_u2i = jnp.where(src_u2i == TRASH, 0, src_u2i)
  dst_u2i = _pad_edges(edge_index_u2i[1])
  src_i2u = jnp.where(_pad_edges(edge_index_i2u[0]) == TRASH, 0,
                      _pad_edges(edge_index_i2u[0]))
  dst_i2u = _pad_edges(edge_index_i2u[1])

  cnts = _counts(jnp.stack([dst_u2i, dst_i2u]), zeros16, ones16)
  cnt_i = cnts[0, :, 0:1]
  cnt_u = cnts[1, :, 0:1]

  h_u = _pre_mlp(x_user, _chunk_w(pre_user_W), pre_user_b.reshape(1, D))
  h_i = _pre_mlp(x_item, _chunk_w(pre_item_W), pre_item_b.reshape(1, D))

  for wl, wr, b_l, wl2, wr2, b_l2 in (
      (l0_u2i_Wl, l0_u2i_Wr, l0_u2i_b, l0_i2u_Wl, l0_i2u_Wr, l0_i2u_b),
      (l1_u2i_Wl, l1_u2i_Wr, l1_u2i_b, l1_i2u_Wl, l1_i2u_Wr, l1_i2u_b),
  ):
    agg_i = _spmm(h_u, src_u2i, dst_u2i, zeros128)
    agg_u = _spmm(h_i, src_i2u, dst_i2u, zeros128)
    n_i = _mid_mlp(agg_i, cnt_i, h_i, _chunk_w(wl), _chunk_w(wr),
                   b_l.reshape(1, D))
    n_u = _mid_mlp(agg_u, cnt_u, h_u, _chunk_w(wl2), _chunk_w(wr2),
                   b_l2.reshape(1, D))
    h_u, h_i = n_u, n_i

  o_u = _post_mlp(h_u, _chunk_w(post_user_W), post_user_b.reshape(1, D))
  o_i = _post_mlp(h_i, _chunk_w(post_item_W), post_item_b.reshape(1, D))
  return o_u, o_i


# speedup vs baseline: 3.7807x; 1.1486x over previous
import functools

import jax
import jax.numpy as jnp
from jax import lax
from jax.experimental import pallas as pl
from jax.experimental.pallas import tpu as pltpu
from jax.experimental.pallas import tpu_sc as plsc

N = 10000
D = 512
NCHUNK = 4
CW = 128
E = 78000
NC, NS = 2, 16
EB = 128
NBATCH = 39
E_PAD = NS * NBATCH * EB
TRASH = N
ACC_ROWS = 10240
ZROWS = ACC_ROWS // NS
OROWS = ACC_ROWS // NS

def _spmm_body(h_hbm, src_hbm, dst_hbm, zeros_hbm, out_hbm,
               acc, src_v, dst_v, gbuf0, gbuf1, sem0, sem1):
  c = lax.axis_index("c")
  s = lax.axis_index("s")
  pltpu.sync_copy(src_hbm.at[s], src_v)
  pltpu.sync_copy(dst_hbm.at[s], dst_v)

  gb = (gbuf0, gbuf1)
  sm = (sem0, sem1)
  for it in range(NCHUNK // NC):
    chunk = c * (NCHUNK // NC) + it
    pltpu.sync_copy(zeros_hbm, gbuf0)
    for k in range(ZROWS // 128):
      acc_rows = acc.at[pl.ds(s * ZROWS + k * 128, 128)]
      pltpu.sync_copy(gbuf0, acc_rows)
    plsc.subcore_barrier()

    cps = [None, None]
    cps[0] = pltpu.async_copy(h_hbm.at[chunk].at[src_v.at[0]], gb[0], sm[0])
    for j in range(NBATCH):
      if j + 1 < NBATCH:
        cps[(j + 1) % 2] = pltpu.async_copy(
            h_hbm.at[chunk].at[src_v.at[j + 1]], gb[(j + 1) % 2],
            sm[(j + 1) % 2])
      cps[j % 2].wait()
      pltpu.sync_copy(gb[j % 2], acc.at[dst_v.at[j]], add=True)
    plsc.subcore_barrier()
    pltpu.sync_copy(acc.at[pl.ds(s * OROWS, OROWS)],
                    out_hbm.at[chunk].at[pl.ds(s * OROWS, OROWS)])
    plsc.subcore_barrier()


@functools.cache
def _make_spmm():
  return pl.kernel(
      _spmm_body,
      out_type=jax.ShapeDtypeStruct((NCHUNK, ACC_ROWS, CW), jnp.float32),
      mesh=plsc.VectorSubcoreMesh(core_axis_name="c", subcore_axis_name="s"),
      scratch_types=[
          pltpu.VMEM_SHARED((ACC_ROWS, CW), jnp.float32),
          pltpu.VMEM((NBATCH, EB), jnp.int32),
          pltpu.VMEM((NBATCH, EB), jnp.int32),
          pltpu.VMEM((EB, CW), jnp.float32),
          pltpu.VMEM((EB, CW), jnp.float32),
          pltpu.SemaphoreType.DMA,
          pltpu.SemaphoreType.DMA,
      ],
  )


def _spmm(h, src_p, dst_p, zeros):
  return _make_spmm()(h, src_p, dst_p, zeros)


CNTW = 128


def _counts_body(dst2_hbm, zeros_hbm, ones_hbm, out_hbm, acc, dst_v, obuf, zbuf):
  c = lax.axis_index("c")
  s = lax.axis_index("s")
  pltpu.sync_copy(dst2_hbm.at[c].at[s], dst_v)
  pltpu.sync_copy(zeros_hbm, zbuf)
  pltpu.sync_copy(ones_hbm, obuf)
  for k in range(ZROWS // 128):
    pltpu.sync_copy(zbuf, acc.at[pl.ds(s * ZROWS + k * 128, 128)])
  plsc.subcore_barrier()

  def body(j, carry):
    pltpu.sync_copy(obuf, acc.at[dst_v.at[j]], add=True)
    return carry

  lax.fori_loop(0, NBATCH, body, 0)
  plsc.subcore_barrier()
  pltpu.sync_copy(acc.at[pl.ds(s * OROWS, OROWS)],
                  out_hbm.at[c].at[pl.ds(s * OROWS, OROWS)])


@functools.cache
def _make_counts():
  return pl.kernel(
      _counts_body,
      out_type=jax.ShapeDtypeStruct((NC, ACC_ROWS, CNTW), jnp.float32),
      mesh=plsc.VectorSubcoreMesh(core_axis_name="c", subcore_axis_name="s"),
      scratch_types=[
          pltpu.VMEM_SHARED((ACC_ROWS, CNTW), jnp.float32),
          pltpu.VMEM((NBATCH, EB), jnp.int32),
          pltpu.VMEM((EB, CNTW), jnp.float32),
          pltpu.VMEM((EB, CNTW), jnp.float32),
      ],
  )


def _counts(dst2, zeros, ones):
  return _make_counts()(dst2, zeros, ones)


BN = 2000
NBLK = N // BN
NEG_SLOPE = 0.01


def _leaky(y):
  return jnp.where(y >= 0, y, NEG_SLOPE * y)


def _mm(x, w):
  return lax.dot_general(x, w, (((1,), (1,)), ((), ())),
                         preferred_element_type=jnp.float32)


def _pre_body(x_ref, w_ref, b_ref, out_ref, acc):
  kc = pl.program_id(1)

  @pl.when(kc == 0)
  def _():
    acc[...] = jnp.zeros_like(acc)

  acc[...] += _mm(x_ref[...], w_ref[0])

  @pl.when(kc == NCHUNK - 1)
  def _():
    y = _leaky(acc[...] + b_ref[...])
    for cc in range(NCHUNK):
      out_ref[cc] = y[:, cc * CW:(cc + 1) * CW]


def _pre_mlp(x, w4, b):
  return pl.pallas_call(
      _pre_body,
      grid=(NBLK, NCHUNK),
      in_specs=[
          pl.BlockSpec((BN, CW), lambda nb, kc: (nb, kc)),
          pl.BlockSpec((1, D, CW), lambda nb, kc: (kc, 0, 0)),
          pl.BlockSpec((1, D), lambda nb, kc: (0, 0)),
      ],
      out_specs=pl.BlockSpec((NCHUNK, BN, CW), lambda nb, kc: (0, nb, 0)),
      out_shape=jax.ShapeDtypeStruct((NCHUNK, N, CW), jnp.float32),
      scratch_shapes=[pltpu.VMEM((BN, D), jnp.float32)],
  )(x, w4, b)


def _mid_body(agg_ref, cnt_ref, h_ref, wl_ref, wr_ref, b_ref, out_ref, acc):
  kc = pl.program_id(1)

  @pl.when(kc == 0)
  def _():
    acc[...] = jnp.zeros_like(acc)

  inv = 1.0 / jnp.maximum(cnt_ref[...], 1.0)
  acc[...] += _mm(agg_ref[0] * inv, wl_ref[0]) + _mm(h_ref[0], wr_ref[0])

  @pl.when(kc == NCHUNK - 1)
  def _():
    y = _leaky(acc[...] + b_ref[...])
    for cc in range(NCHUNK):
      out_ref[cc] = y[:, cc * CW:(cc + 1) * CW]


def _mid_mlp(agg, cnt, h, wl4, wr4, b):
  return pl.pallas_call(
      _mid_body,
      grid=(NBLK, NCHUNK),
      in_specs=[
          pl.BlockSpec((1, BN, CW), lambda nb, kc: (kc, nb, 0)),
          pl.BlockSpec((BN, 1), lambda nb, kc: (nb, 0)),
          pl.BlockSpec((1, BN, CW), lambda nb, kc: (kc, nb, 0)),
          pl.BlockSpec((1, D, CW), lambda nb, kc: (kc, 0, 0)),
          pl.BlockSpec((1, D, CW), lambda nb, kc: (kc, 0, 0)),
          pl.BlockSpec((1, D), lambda nb, kc: (0, 0)),
      ],
      out_specs=pl.BlockSpec((NCHUNK, BN, CW), lambda nb, kc: (0, nb, 0)),
      out_shape=jax.ShapeDtypeStruct((NCHUNK, N, CW), jnp.float32),
      scratch_shapes=[pltpu.VMEM((BN, D), jnp.float32)],
  )(agg, cnt, h, wl4, wr4, b)


def _post_body(h_ref, w_ref, b_ref, out_ref, acc):
  kc = pl.program_id(1)

  @pl.when(kc == 0)
  def _():
    acc[...] = jnp.zeros_like(acc)

  acc[...] += _mm(h_ref[0], w_ref[0])

  @pl.when(kc == NCHUNK - 1)
  def _():
    y = acc[...] + b_ref[...]
    nrm = jnp.maximum(jnp.sqrt(jnp.sum(y * y, axis=1, keepdims=True)), 1e-12)
    out_ref[...] = y / nrm


def _post_mlp(h, w4, b):
  return pl.pallas_call(
      _post_body,
      grid=(NBLK, NCHUNK),
      in_specs=[
          pl.BlockSpec((1, BN, CW), lambda nb, kc: (kc, nb, 0)),
          pl.BlockSpec((1, D, CW), lambda nb, kc: (kc, 0, 0)),
          pl.BlockSpec((1, D), lambda nb, kc: (0, 0)),
      ],
      out_specs=pl.BlockSpec((BN, D), lambda nb, kc: (nb, 0)),
      out_shape=jax.ShapeDtypeStruct((N, D), jnp.float32),
      scratch_shapes=[pltpu.VMEM((BN, D), jnp.float32)],
  )(h, w4, b)


def _chunk_w(w):
  return jnp.transpose(w.reshape(D, NCHUNK, CW), (1, 0, 2))


def _pad_edges(e):
  return jnp.concatenate(
      [e.astype(jnp.int32), jnp.full((E_PAD - E,), TRASH, jnp.int32)]
  ).reshape(NS, NBATCH, EB)


def kernel(x_user, x_item, edge_index_u2i, edge_index_i2u,
           pre_user_W, pre_user_b, pre_item_W, pre_item_b,
           post_user_W, post_user_b, post_item_W, post_item_b,
           l0_u2i_Wl, l0_u2i_Wr, l0_u2i_b,
           l0_i2u_Wl, l0_i2u_Wr, l0_i2u_b,
           l1_u2i_Wl, l1_u2i_Wr, l1_u2i_b,
           l1_i2u_Wl, l1_i2u_Wr, l1_i2u_b):
  zeros128 = jnp.zeros((EB, CW), jnp.float32)
  zeros16 = jnp.zeros((EB, CNTW), jnp.float32)
  ones16 = jnp.ones((EB, CNTW), jnp.float32)

  src_u2i = _pad_edges(edge_index_u2i[0])
  src_u2i = jnp.where(src_u2i == TRASH, 0, src_u2i)
  dst_u2i = _pad_edges(edge_index_u2i[1])
  src_i2u = jnp.where(_pad_edges(edge_index_i2u[0]) == TRASH, 0,
                      _pad_edges(edge_index_i2u[0]))
  dst_i2u = _pad_edges(edge_index_i2u[1])

  cnts = _counts(jnp.stack([dst_u2i, dst_i2u]), zeros16, ones16)
  cnt_i = cnts[0, :, 0:1]
  cnt_u = cnts[1, :, 0:1]

  h_u = _pre_mlp(x_user, _chunk_w(pre_user_W), pre_user_b.reshape(1, D))
  h_i = _pre_mlp(x_item, _chunk_w(pre_item_W), pre_item_b.reshape(1, D))

  for wl, wr, b_l, wl2, wr2, b_l2 in (
      (l0_u2i_Wl, l0_u2i_Wr, l0_u2i_b, l0_i2u_Wl, l0_i2u_Wr, l0_i2u_b),
      (l1_u2i_Wl, l1_u2i_Wr, l1_u2i_b, l1_i2u_Wl, l1_i2u_Wr, l1_i2u_b),
  ):
    agg_i = _spmm(h_u, src_u2i, dst_u2i, zeros128)
    agg_u = _spmm(h_i, src_i2u, dst_i2u, zeros128)
    n_i = _mid_mlp(agg_i, cnt_i, h_i, _chunk_w(wl), _chunk_w(wr),
                   b_l.reshape(1, D))
    n_u = _mid_mlp(agg_u, cnt_u, h_u, _chunk_w(wl2), _chunk_w(wr2),
                   b_l2.reshape(1, D))
    h_u, h_i = n_u, n_i

  o_u = _post_mlp(h_u, _chunk_w(post_user_W), post_user_b.reshape(1, D))
  o_i = _post_mlp(h_i, _chunk_w(post_item_W), post_item_b.reshape(1, D))
  return o_u, o_i


# speedup vs baseline: 3.8436x; 1.0166x over previous
import functools

import jax
import jax.numpy as jnp
from jax import lax
from jax.experimental import pallas as pl
from jax.experimental.pallas import tpu as pltpu
from jax.experimental.pallas import tpu_sc as plsc

N = 10000
D = 512
NCHUNK = 4
CW = 128
E = 78000
NC, NS = 2, 16
EB = 128
NBATCH = 39
E_PAD = NS * NBATCH * EB
TRASH = N
ACC_ROWS = 10240
ZROWS = ACC_ROWS // NS
OROWS = ACC_ROWS // NS

def _spmm_body(h_hbm, src_hbm, dst_hbm, zeros_hbm, out_hbm,
               acc, src_v, dst_v, gbuf0, gbuf1, sem0, sem1, ssem0, ssem1):
  c = lax.axis_index("c")
  s = lax.axis_index("s")
  pltpu.sync_copy(src_hbm.at[s], src_v)
  pltpu.sync_copy(dst_hbm.at[s], dst_v)

  gb = (gbuf0, gbuf1)
  sm = (sem0, sem1)
  ssm = (ssem0, ssem1)
  for it in range(NCHUNK // NC):
    chunk = c * (NCHUNK // NC) + it
    pltpu.sync_copy(zeros_hbm, gbuf0)
    for k in range(ZROWS // 128):
      acc_rows = acc.at[pl.ds(s * ZROWS + k * 128, 128)]
      pltpu.sync_copy(gbuf0, acc_rows)
    plsc.subcore_barrier()

    cps = [None, None]
    scs = [None, None]
    cps[0] = pltpu.async_copy(h_hbm.at[chunk].at[src_v.at[0]], gb[0], sm[0])
    for j in range(NBATCH):
      if j + 1 < NBATCH:
        if scs[(j + 1) % 2] is not None:
          scs[(j + 1) % 2].wait()
          scs[(j + 1) % 2] = None
        cps[(j + 1) % 2] = pltpu.async_copy(
            h_hbm.at[chunk].at[src_v.at[j + 1]], gb[(j + 1) % 2],
            sm[(j + 1) % 2])
      cps[j % 2].wait()
      scs[j % 2] = pltpu.async_copy(gb[j % 2], acc.at[dst_v.at[j]],
                                    ssm[j % 2], add=True)
    for sc in scs:
      if sc is not None:
        sc.wait()
    plsc.subcore_barrier()
    pltpu.sync_copy(acc.at[pl.ds(s * OROWS, OROWS)],
                    out_hbm.at[chunk].at[pl.ds(s * OROWS, OROWS)])
    plsc.subcore_barrier()


@functools.cache
def _make_spmm():
  return pl.kernel(
      _spmm_body,
      out_type=jax.ShapeDtypeStruct((NCHUNK, ACC_ROWS, CW), jnp.float32),
      mesh=plsc.VectorSubcoreMesh(core_axis_name="c", subcore_axis_name="s"),
      scratch_types=[
          pltpu.VMEM_SHARED((ACC_ROWS, CW), jnp.float32),
          pltpu.VMEM((NBATCH, EB), jnp.int32),
          pltpu.VMEM((NBATCH, EB), jnp.int32),
          pltpu.VMEM((EB, CW), jnp.float32),
          pltpu.VMEM((EB, CW), jnp.float32),
          pltpu.SemaphoreType.DMA,
          pltpu.SemaphoreType.DMA,
          pltpu.SemaphoreType.DMA,
          pltpu.SemaphoreType.DMA,
      ],
  )


def _spmm(h, src_p, dst_p, zeros):
  return _make_spmm()(h, src_p, dst_p, zeros)


CNTW = 128


def _counts_body(dst2_hbm, zeros_hbm, ones_hbm, out_hbm, acc, dst_v, obuf, zbuf):
  c = lax.axis_index("c")
  s = lax.axis_index("s")
  pltpu.sync_copy(dst2_hbm.at[c].at[s], dst_v)
  pltpu.sync_copy(zeros_hbm, zbuf)
  pltpu.sync_copy(ones_hbm, obuf)
  for k in range(ZROWS // 128):
    pltpu.sync_copy(zbuf, acc.at[pl.ds(s * ZROWS + k * 128, 128)])
  plsc.subcore_barrier()

  def body(j, carry):
    pltpu.sync_copy(obuf, acc.at[dst_v.at[j]], add=True)
    return carry

  lax.fori_loop(0, NBATCH, body, 0)
  plsc.subcore_barrier()
  pltpu.sync_copy(acc.at[pl.ds(s * OROWS, OROWS)],
                  out_hbm.at[c].at[pl.ds(s * OROWS, OROWS)])


@functools.cache
def _make_counts():
  return pl.kernel(
      _counts_body,
      out_type=jax.ShapeDtypeStruct((NC, ACC_ROWS, CNTW), jnp.float32),
      mesh=plsc.VectorSubcoreMesh(core_axis_name="c", subcore_axis_name="s"),
      scratch_types=[
          pltpu.VMEM_SHARED((ACC_ROWS, CNTW), jnp.float32),
          pltpu.VMEM((NBATCH, EB), jnp.int32),
          pltpu.VMEM((EB, CNTW), jnp.float32),
          pltpu.VMEM((EB, CNTW), jnp.float32),
      ],
  )


def _counts(dst2, zeros, ones):
  return _make_counts()(dst2, zeros, ones)


BN = 2000
NBLK = N // BN
NEG_SLOPE = 0.01


def _leaky(y):
  return jnp.where(y >= 0, y, NEG_SLOPE * y)


def _mm(x, w):
  return lax.dot_general(x, w, (((1,), (1,)), ((), ())),
                         preferred_element_type=jnp.float32)


def _pre_body(x_ref, w_ref, b_ref, out_ref, acc):
  kc = pl.program_id(1)

  @pl.when(kc == 0)
  def _():
    acc[...] = jnp.zeros_like(acc)

  acc[...] += _mm(x_ref[...], w_ref[0])

  @pl.when(kc == NCHUNK - 1)
  def _():
    y = _leaky(acc[...] + b_ref[...])
    for cc in range(NCHUNK):
      out_ref[cc] = y[:, cc * CW:(cc + 1) * CW]


def _pre_mlp(x, w4, b):
  return pl.pallas_call(
      _pre_body,
      grid=(NBLK, NCHUNK),
      in_specs=[
          pl.BlockSpec((BN, CW), lambda nb, kc: (nb, kc)),
          pl.BlockSpec((1, D, CW), lambda nb, kc: (kc, 0, 0)),
          pl.BlockSpec((1, D), lambda nb, kc: (0, 0)),
      ],
      out_specs=pl.BlockSpec((NCHUNK, BN, CW), lambda nb, kc: (0, nb, 0)),
      out_shape=jax.ShapeDtypeStruct((NCHUNK, N, CW), jnp.float32),
      scratch_shapes=[pltpu.VMEM((BN, D), jnp.float32)],
  )(x, w4, b)


def _mid_body(agg_ref, cnt_ref, h_ref, wl_ref, wr_ref, b_ref, out_ref, acc):
  kc = pl.program_id(1)

  @pl.when(kc == 0)
  def _():
    acc[...] = jnp.zeros_like(acc)

  inv = 1.0 / jnp.maximum(cnt_ref[...], 1.0)
  acc[...] += _mm(agg_ref[0] * inv, wl_ref[0]) + _mm(h_ref[0], wr_ref[0])

  @pl.when(kc == NCHUNK - 1)
  def _():
    y = _leaky(acc[...] + b_ref[...])
    for cc in range(NCHUNK):
      out_ref[cc] = y[:, cc * CW:(cc + 1) * CW]


def _mid_mlp(agg, cnt, h, wl4, wr4, b):
  return pl.pallas_call(
      _mid_body,
      grid=(NBLK, NCHUNK),
      in_specs=[
          pl.BlockSpec((1, BN, CW), lambda nb, kc: (kc, nb, 0)),
          pl.BlockSpec((BN, 1), lambda nb, kc: (nb, 0)),
          pl.BlockSpec((1, BN, CW), lambda nb, kc: (kc, nb, 0)),
          pl.BlockSpec((1, D, CW), lambda nb, kc: (kc, 0, 0)),
          pl.BlockSpec((1, D, CW), lambda nb, kc: (kc, 0, 0)),
          pl.BlockSpec((1, D), lambda nb, kc: (0, 0)),
      ],
      out_specs=pl.BlockSpec((NCHUNK, BN, CW), lambda nb, kc: (0, nb, 0)),
      out_shape=jax.ShapeDtypeStruct((NCHUNK, N, CW), jnp.float32),
      scratch_shapes=[pltpu.VMEM((BN, D), jnp.float32)],
  )(agg, cnt, h, wl4, wr4, b)


def _post_body(h_ref, w_ref, b_ref, out_ref, acc):
  kc = pl.program_id(1)

  @pl.when(kc == 0)
  def _():
    acc[...] = jnp.zeros_like(acc)

  acc[...] += _mm(h_ref[0], w_ref[0])

  @pl.when(kc == NCHUNK - 1)
  def _():
    y = acc[...] + b_ref[...]
    nrm = jnp.maximum(jnp.sqrt(jnp.sum(y * y, axis=1, keepdims=True)), 1e-12)
    out_ref[...] = y / nrm


def _post_mlp(h, w4, b):
  return pl.pallas_call(
      _post_body,
      grid=(NBLK, NCHUNK),
      in_specs=[
          pl.BlockSpec((1, BN, CW), lambda nb, kc: (kc, nb, 0)),
          pl.BlockSpec((1, D, CW), lambda nb, kc: (kc, 0, 0)),
          pl.BlockSpec((1, D), lambda nb, kc: (0, 0)),
      ],
      out_specs=pl.BlockSpec((BN, D), lambda nb, kc: (nb, 0)),
      out_shape=jax.ShapeDtypeStruct((N, D), jnp.float32),
      scratch_shapes=[pltpu.VMEM((BN, D), jnp.float32)],
  )(h, w4, b)


def _chunk_w(w):
  return jnp.transpose(w.reshape(D, NCHUNK, CW), (1, 0, 2))


def _pad_edges(e):
  return jnp.concatenate(
      [e.astype(jnp.int32), jnp.full((E_PAD - E,), TRASH, jnp.int32)]
  ).reshape(NS, NBATCH, EB)


def kernel(x_user, x_item, edge_index_u2i, edge_index_i2u,
           pre_user_W, pre_user_b, pre_item_W, pre_item_b,
           post_user_W, post_user_b, post_item_W, post_item_b,
           l0_u2i_Wl, l0_u2i_Wr, l0_u2i_b,
           l0_i2u_Wl, l0_i2u_Wr, l0_i2u_b,
           l1_u2i_Wl, l1_u2i_Wr, l1_u2i_b,
           l1_i2u_Wl, l1_i2u_Wr, l1_i2u_b):
  zeros128 = jnp.zeros((EB, CW), jnp.float32)
  zeros16 = jnp.zeros((EB, CNTW), jnp.float32)
  ones16 = jnp.ones((EB, CNTW), jnp.float32)

  src_u2i = _pad_edges(edge_index_u2i[0])
  src_u2i = jnp.where(src_u2i == TRASH, 0, src_u2i)
  dst_u2i = _pad_edges(edge_index_u2i[1])
  src_i2u = jnp.where(_pad_edges(edge_index_i2u[0]) == TRASH, 0,
                      _pad_edges(edge_index_i2u[0]))
  dst_i2u = _pad_edges(edge_index_i2u[1])

  cnts = _counts(jnp.stack([dst_u2i, dst_i2u]), zeros16, ones16)
  cnt_i = cnts[0, :, 0:1]
  cnt_u = cnts[1, :, 0:1]

  h_u = _pre_mlp(x_user, _chunk_w(pre_user_W), pre_user_b.reshape(1, D))
  h_i = _pre_mlp(x_item, _chunk_w(pre_item_W), pre_item_b.reshape(1, D))

  for wl, wr, b_l, wl2, wr2, b_l2 in (
      (l0_u2i_Wl, l0_u2i_Wr, l0_u2i_b, l0_i2u_Wl, l0_i2u_Wr, l0_i2u_b),
      (l1_u2i_Wl, l1_u2i_Wr, l1_u2i_b, l1_i2u_Wl, l1_i2u_Wr, l1_i2u_b),
  ):
    agg_i = _spmm(h_u, src_u2i, dst_u2i, zeros128)
    agg_u = _spmm(h_i, src_i2u, dst_i2u, zeros128)
    n_i = _mid_mlp(agg_i, cnt_i, h_i, _chunk_w(wl), _chunk_w(wr),
                   b_l.reshape(1, D))
    n_u = _mid_mlp(agg_u, cnt_u, h_u, _chunk_w(wl2), _chunk_w(wr2),
                   b_l2.reshape(1, D))
    h_u, h_i = n_u, n_i

  o_u = _post_mlp(h_u, _chunk_w(post_user_W), post_user_b.reshape(1, D))
  o_i = _post_mlp(h_i, _chunk_w(post_item_W), post_item_b.reshape(1, D))
  return o_u, o_i
